# baseline (speedup 1.0000x reference)
import functools
import numpy as np
import jax
import jax.numpy as jnp
from jax import lax
from jax.experimental import pallas as pl
from jax.experimental.pallas import tpu as pltpu
from jax.experimental.pallas import tpu_sc as plsc


def _tc_self(node_input, W_self):
    Nn, D = node_input.shape
    DT = W_self.shape[1]
    DOUT = DT - D
    BN = 1000
    assert Nn % BN == 0

    def body(x_ref, w_ref, nf_ref, nso_ref):
        tmp = jnp.dot(x_ref[...], w_ref[...], preferred_element_type=jnp.float32)
        nf_ref[...] = tmp[:, :D]
        nso_ref[...] = tmp[:, D:]

    return pl.pallas_call(
        body,
        grid=(Nn // BN,),
        in_specs=[
            pl.BlockSpec((BN, D), lambda i: (i, 0)),
            pl.BlockSpec((D, DT), lambda i: (0, 0)),
        ],
        out_specs=[
            pl.BlockSpec((BN, D), lambda i: (i, 0)),
            pl.BlockSpec((BN, DOUT), lambda i: (i, 0)),
        ],
        out_shape=[
            jax.ShapeDtypeStruct((Nn, D), jnp.float32),
            jax.ShapeDtypeStruct((Nn, DOUT), jnp.float32),
        ],
    )(node_input, W_self)


def _tc_mix(edge_attr, edge_scalar_attr, W1P, W2R, T, Wflat,
            e_lo=0, e_cnt=None):
    DE, E = edge_attr.shape
    if e_cnt is None:
        e_cnt = E
    HW = W1P.shape[1]
    HD = W2R.shape[1]
    D = Wflat.shape[1]
    BE = 3200
    assert e_cnt % BE == 0 and e_lo % BE == 0
    blk0 = e_lo // BE

    def dn(a, b):
        return lax.dot_general(a, b, (((0,), (0,)), ((), ())),
                               preferred_element_type=jnp.float32)

    def body(attr_ref, sattr_ref, w1_ref, w2_ref, t_ref, wf_ref, out_ref):
        g1 = jax.nn.gelu(dn(w1_ref[...], sattr_ref[...]))
        g2 = jax.nn.gelu(dn(w2_ref[...], g1.astype(jnp.bfloat16)))
        P = g2 * dn(t_ref[...], attr_ref[...])
        out_ref[...] = dn(P.astype(jnp.bfloat16), wf_ref[...])

    return pl.pallas_call(
        body,
        grid=(e_cnt // BE,),
        in_specs=[
            pl.BlockSpec((DE, BE), lambda i: (0, i + blk0)),
            pl.BlockSpec((DE, BE), lambda i: (0, i + blk0)),
            pl.BlockSpec((DE, HW), lambda i: (0, 0)),
            pl.BlockSpec((HW, HD), lambda i: (0, 0)),
            pl.BlockSpec((DE, HD), lambda i: (0, 0)),
            pl.BlockSpec((HD, D), lambda i: (0, 0)),
        ],
        out_specs=pl.BlockSpec((BE, D), lambda i: (i, 0)),
        out_shape=jax.ShapeDtypeStruct((e_cnt, D), jnp.float32),
    )(edge_attr, edge_scalar_attr, W1P, W2R, T, Wflat)


def _sc_gather_mul_scatter(node_features, mix, edge_src, edge_dst):
    Nn, D = node_features.shape
    E = edge_src.shape[0]
    K = 128
    NC, NS = 2, 16
    NW = NC * NS
    total_chunks = E // K
    base_chunks = total_chunks // NW
    extra = total_chunks % NW
    L = 16
    base_rows = (Nn // NS) // 8 * 8
    tail_rows = Nn - NS * base_rows
    assert E % K == 0 and D % L == 0
    assert base_rows % 8 == 0 and tail_rows % 8 == 0 and Nn % 8 == 0

    mesh = plsc.VectorSubcoreMesh(core_axis_name="c", subcore_axis_name="s")

    @functools.partial(
        pl.kernel,
        out_type=jax.ShapeDtypeStruct((NC, Nn, D), jnp.float32),
        mesh=mesh,
        scratch_types=[
            pltpu.VMEM((2, K), jnp.int32),
            pltpu.VMEM((2, K), jnp.int32),
            pltpu.VMEM((2, K, D), jnp.float32),
            pltpu.VMEM((K, D), jnp.float32),
            pltpu.VMEM_SHARED((Nn, D), jnp.float32),
            pltpu.SemaphoreType.DMA((2,)),
            pltpu.SemaphoreType.DMA((2,)),
            pltpu.SemaphoreType.DMA,
            pltpu.SemaphoreType.DMA((2,)),
        ],
    )
    def k(nf_hbm, mix_hbm, src_hbm, dst_hbm, out_hbm,
          src_v, dst_v, rows_v, mix_v, agg_sh,
          sem_src, sem_dst, sem_mix, sem_g):
        c = lax.axis_index("c")
        s = lax.axis_index("s")
        wid = s * NC + c
        my_chunks = base_chunks + jnp.where(wid < extra, 1, 0)

        def issue_front(j, b):
            e0 = (j * NW + wid) * K
            pltpu.async_copy(src_hbm.at[pl.ds(e0, K)], src_v.at[b],
                             sem_src.at[b])
            pltpu.async_copy(dst_hbm.at[pl.ds(e0, K)], dst_v.at[b],
                             sem_dst.at[b])

        def issue_mix(j):
            e0 = (j * NW + wid) * K
            pltpu.async_copy(mix_hbm.at[pl.ds(e0, K)], mix_v, sem_mix)

        def wait_src_and_gather(b):
            pltpu.make_async_copy(src_hbm.at[pl.ds(0, K)], src_v.at[b],
                                  sem_src.at[b]).wait()
            pltpu.async_copy(nf_hbm.at[src_v.at[b]], rows_v.at[b],
                             sem_g.at[b])

        def process(j, b):
            pltpu.make_async_copy(nf_hbm.at[src_v.at[b]], rows_v.at[b],
                                  sem_g.at[b]).wait()
            pltpu.make_async_copy(mix_hbm.at[pl.ds(0, K)], mix_v,
                                  sem_mix).wait()

            @plsc.parallel_loop(0, K, 1, unroll=2)
            def _(r):
                for l in range(D // L):
                    sl = pl.ds(l * L, L)
                    rows_v[b, r, sl] = rows_v[b, r, sl] * mix_v[r, sl]

            @pl.when(j + 1 < my_chunks)
            def _():
                issue_mix(j + 1)

            pltpu.make_async_copy(dst_hbm.at[pl.ds(0, K)], dst_v.at[b],
                                  sem_dst.at[b]).wait()
            pltpu.sync_copy(rows_v.at[b], agg_sh.at[dst_v.at[b]], add=True)

        zero = jnp.zeros((L,), jnp.float32)

        def zrow(r, _):
            for l in range(D // L):
                rows_v[0, r, pl.ds(l * L, L)] = zero
            return 0
        lax.fori_loop(0, K, zrow, 0, unroll=False)
        nfull, rem = divmod(base_rows, K)
        for t in range(nfull):
            pltpu.sync_copy(rows_v.at[0],
                            agg_sh.at[pl.ds(s * base_rows + t * K, K)])
        if rem:
            pltpu.sync_copy(rows_v.at[0, pl.ds(0, rem)],
                            agg_sh.at[pl.ds(s * base_rows + nfull * K, rem)])

        @pl.when(s == NS - 1)
        def _():
            pltpu.sync_copy(rows_v.at[0, pl.ds(0, tail_rows)],
                            agg_sh.at[pl.ds(NS * base_rows, tail_rows)])
        plsc.subcore_barrier()

        issue_front(0, 0)
        issue_front(1, 1)
        issue_mix(0)
        wait_src_and_gather(0)

        def step(j, b):
            @pl.when(j + 1 < my_chunks)
            def _():
                wait_src_and_gather(1 - b)
            process(j, b)

            @pl.when(j + 2 < my_chunks)
            def _():
                issue_front(j + 2, b)

        def pair(g, _):
            for b in range(2):
                step(2 * g + b, b)
            return 0
        lax.fori_loop(0, base_chunks // 2, pair, 0, unroll=False)

        for jt in range(2 * (base_chunks // 2), base_chunks + 1):
            @pl.when(jt < my_chunks)
            def _(jt=jt):
                step(jt, jt % 2)

        plsc.subcore_barrier()
        pltpu.sync_copy(agg_sh.at[pl.ds(s * base_rows, base_rows)],
                        out_hbm.at[c, pl.ds(s * base_rows, base_rows)])

        @pl.when(s == NS - 1)
        def _():
            pltpu.sync_copy(agg_sh.at[pl.ds(NS * base_rows, tail_rows)],
                            out_hbm.at[c, pl.ds(NS * base_rows, tail_rows)])

    return k(node_features, mix, edge_src, edge_dst)


def _tc_final(node_self_out, aggs, W_out_scaled, cos_mix):
    Nn, DOUT = node_self_out.shape
    D = aggs[0].shape[2]
    BN = 1000
    assert Nn % BN == 0

    def body(nso_ref, *rest):
        agg_refs, w_ref, out_ref = rest[:-2], rest[-2], rest[-1]
        agg = agg_refs[0][0] + agg_refs[0][1]
        for a in agg_refs[1:]:
            agg = agg + a[0] + a[1]
        conv = jnp.dot(agg, w_ref[...], preferred_element_type=jnp.float32)
        out_ref[...] = cos_mix * nso_ref[...] + conv

    return pl.pallas_call(
        body,
        grid=(Nn // BN,),
        in_specs=[
            pl.BlockSpec((BN, DOUT), lambda i: (i, 0)),
        ] + [
            pl.BlockSpec((2, BN, D), lambda i: (0, i, 0)) for _ in aggs
        ] + [
            pl.BlockSpec((D, DOUT), lambda i: (0, 0)),
        ],
        out_specs=pl.BlockSpec((BN, DOUT), lambda i: (i, 0)),
        out_shape=jax.ShapeDtypeStruct((Nn, DOUT), jnp.float32),
    )(node_self_out, *aggs, W_out_scaled)


def kernel(node_input, edge_src, edge_dst, edge_attr, edge_scalar_attr,
           W_self, W_mlp1, W_mlp2, W_tp, W_out):
    H, D, DE = W_tp.shape
    NUM_NEIGHBORS = 32.0
    MIXING_ANGLE = 0.39269908169872414
    c = np.cos(MIXING_ANGLE)
    s = np.sin(MIXING_ANGLE)

    Wflat = (W_tp.transpose(0, 2, 1) / np.sqrt(H)).reshape(H * DE, D)
    W_out_scaled = W_out * (s / np.sqrt(NUM_NEIGHBORS))
    HW = 128
    W1P = jnp.pad(W_mlp1, ((0, 0), (0, HW - H)))
    W2R = jnp.pad(jnp.repeat(W_mlp2, DE, axis=1),
                  ((0, HW - H), (0, 0))).astype(jnp.bfloat16)
    T = jnp.tile(jnp.eye(DE, dtype=jnp.float32), (1, H))
    Wflat = Wflat.astype(jnp.bfloat16)

    node_features, node_self_out = _tc_self(node_input, W_self)
    E = edge_src.shape[0]
    NPH = 4
    EP = E // NPH
    partials = []
    for ph in range(NPH):
        lo = ph * EP
        mix_ph = _tc_mix(edge_attr.T, edge_scalar_attr.T, W1P, W2R, T, Wflat,
                         e_lo=lo, e_cnt=EP)
        partials.append(
            _sc_gather_mul_scatter(node_features, mix_ph,
                                   lax.slice(edge_src, (lo,), (lo + EP,)),
                                   lax.slice(edge_dst, (lo,), (lo + EP,))))
    return _tc_final(node_self_out, partials, W_out_scaled, c)


# speedup vs baseline: 1.0035x; 1.0035x over previous
import functools
import numpy as np
import jax
import jax.numpy as jnp
from jax import lax
from jax.experimental import pallas as pl
from jax.experimental.pallas import tpu as pltpu
from jax.experimental.pallas import tpu_sc as plsc


def _tc_self(node_input, W_self):
    Nn, D = node_input.shape
    DT = W_self.shape[1]
    DOUT = DT - D
    BN = 1000
    assert Nn % BN == 0

    def body(x_ref, w_ref, nf_ref, nso_ref):
        tmp = jnp.dot(x_ref[...], w_ref[...], preferred_element_type=jnp.float32)
        nf_ref[...] = tmp[:, :D]
        nso_ref[...] = tmp[:, D:]

    return pl.pallas_call(
        body,
        grid=(Nn // BN,),
        in_specs=[
            pl.BlockSpec((BN, D), lambda i: (i, 0)),
            pl.BlockSpec((D, DT), lambda i: (0, 0)),
        ],
        out_specs=[
            pl.BlockSpec((BN, D), lambda i: (i, 0)),
            pl.BlockSpec((BN, DOUT), lambda i: (i, 0)),
        ],
        out_shape=[
            jax.ShapeDtypeStruct((Nn, D), jnp.float32),
            jax.ShapeDtypeStruct((Nn, DOUT), jnp.float32),
        ],
    )(node_input, W_self)


def _tc_mix(edge_attr, edge_scalar_attr, W1P, W2R, T, Wflat,
            e_lo=0, e_cnt=None):
    DE, E = edge_attr.shape
    if e_cnt is None:
        e_cnt = E
    HW = W1P.shape[1]
    HD = W2R.shape[1]
    D = Wflat.shape[1]
    BE = 3200
    assert e_cnt % BE == 0 and e_lo % BE == 0
    blk0 = e_lo // BE

    def dn(a, b):
        return lax.dot_general(a, b, (((0,), (0,)), ((), ())),
                               preferred_element_type=jnp.float32)

    def body(attr_ref, sattr_ref, w1_ref, w2_ref, t_ref, wf_ref, out_ref):
        g1 = jax.nn.gelu(dn(w1_ref[...], sattr_ref[...]))
        g2 = jax.nn.gelu(dn(w2_ref[...], g1))
        P = g2 * dn(t_ref[...], attr_ref[...])
        out_ref[...] = dn(P, wf_ref[...])

    return pl.pallas_call(
        body,
        grid=(e_cnt // BE,),
        in_specs=[
            pl.BlockSpec((DE, BE), lambda i: (0, i + blk0)),
            pl.BlockSpec((DE, BE), lambda i: (0, i + blk0)),
            pl.BlockSpec((DE, HW), lambda i: (0, 0)),
            pl.BlockSpec((HW, HD), lambda i: (0, 0)),
            pl.BlockSpec((DE, HD), lambda i: (0, 0)),
            pl.BlockSpec((HD, D), lambda i: (0, 0)),
        ],
        out_specs=pl.BlockSpec((BE, D), lambda i: (i, 0)),
        out_shape=jax.ShapeDtypeStruct((e_cnt, D), jnp.float32),
    )(edge_attr, edge_scalar_attr, W1P, W2R, T, Wflat)


def _sc_gather_mul_scatter(node_features, mix, edge_src, edge_dst):
    Nn, D = node_features.shape
    E = edge_src.shape[0]
    K = 128
    NC, NS = 2, 16
    NW = NC * NS
    total_chunks = E // K
    base_chunks = total_chunks // NW
    extra = total_chunks % NW
    L = 16
    base_rows = (Nn // NS) // 8 * 8
    tail_rows = Nn - NS * base_rows
    assert E % K == 0 and D % L == 0
    assert base_rows % 8 == 0 and tail_rows % 8 == 0 and Nn % 8 == 0

    mesh = plsc.VectorSubcoreMesh(core_axis_name="c", subcore_axis_name="s")

    @functools.partial(
        pl.kernel,
        out_type=jax.ShapeDtypeStruct((NC, Nn, D), jnp.float32),
        mesh=mesh,
        scratch_types=[
            pltpu.VMEM((2, K), jnp.int32),
            pltpu.VMEM((2, K), jnp.int32),
            pltpu.VMEM((2, K, D), jnp.float32),
            pltpu.VMEM((K, D), jnp.float32),
            pltpu.VMEM_SHARED((Nn, D), jnp.float32),
            pltpu.SemaphoreType.DMA((2,)),
            pltpu.SemaphoreType.DMA((2,)),
            pltpu.SemaphoreType.DMA,
            pltpu.SemaphoreType.DMA((2,)),
        ],
    )
    def k(nf_hbm, mix_hbm, src_hbm, dst_hbm, out_hbm,
          src_v, dst_v, rows_v, mix_v, agg_sh,
          sem_src, sem_dst, sem_mix, sem_g):
        c = lax.axis_index("c")
        s = lax.axis_index("s")
        wid = s * NC + c
        my_chunks = base_chunks + jnp.where(wid < extra, 1, 0)

        def issue_front(j, b):
            e0 = (j * NW + wid) * K
            pltpu.async_copy(src_hbm.at[pl.ds(e0, K)], src_v.at[b],
                             sem_src.at[b])
            pltpu.async_copy(dst_hbm.at[pl.ds(e0, K)], dst_v.at[b],
                             sem_dst.at[b])

        def issue_mix(j):
            e0 = (j * NW + wid) * K
            pltpu.async_copy(mix_hbm.at[pl.ds(e0, K)], mix_v, sem_mix)

        def wait_src_and_gather(b):
            pltpu.make_async_copy(src_hbm.at[pl.ds(0, K)], src_v.at[b],
                                  sem_src.at[b]).wait()
            pltpu.async_copy(nf_hbm.at[src_v.at[b]], rows_v.at[b],
                             sem_g.at[b])

        def process(j, b):
            pltpu.make_async_copy(nf_hbm.at[src_v.at[b]], rows_v.at[b],
                                  sem_g.at[b]).wait()
            pltpu.make_async_copy(mix_hbm.at[pl.ds(0, K)], mix_v,
                                  sem_mix).wait()

            @plsc.parallel_loop(0, K, 1, unroll=2)
            def _(r):
                for l in range(D // L):
                    sl = pl.ds(l * L, L)
                    rows_v[b, r, sl] = rows_v[b, r, sl] * mix_v[r, sl]

            @pl.when(j + 1 < my_chunks)
            def _():
                issue_mix(j + 1)

            pltpu.make_async_copy(dst_hbm.at[pl.ds(0, K)], dst_v.at[b],
                                  sem_dst.at[b]).wait()
            pltpu.sync_copy(rows_v.at[b], agg_sh.at[dst_v.at[b]], add=True)

        zero = jnp.zeros((L,), jnp.float32)

        def zrow(r, _):
            for l in range(D // L):
                rows_v[0, r, pl.ds(l * L, L)] = zero
            return 0
        lax.fori_loop(0, K, zrow, 0, unroll=False)
        nfull, rem = divmod(base_rows, K)
        for t in range(nfull):
            pltpu.sync_copy(rows_v.at[0],
                            agg_sh.at[pl.ds(s * base_rows + t * K, K)])
        if rem:
            pltpu.sync_copy(rows_v.at[0, pl.ds(0, rem)],
                            agg_sh.at[pl.ds(s * base_rows + nfull * K, rem)])

        @pl.when(s == NS - 1)
        def _():
            pltpu.sync_copy(rows_v.at[0, pl.ds(0, tail_rows)],
                            agg_sh.at[pl.ds(NS * base_rows, tail_rows)])
        plsc.subcore_barrier()

        issue_front(0, 0)
        issue_front(1, 1)
        issue_mix(0)
        wait_src_and_gather(0)

        def step(j, b):
            @pl.when(j + 1 < my_chunks)
            def _():
                wait_src_and_gather(1 - b)
            process(j, b)

            @pl.when(j + 2 < my_chunks)
            def _():
                issue_front(j + 2, b)

        def pair(g, _):
            for b in range(2):
                step(2 * g + b, b)
            return 0
        lax.fori_loop(0, base_chunks // 2, pair, 0, unroll=False)

        for jt in range(2 * (base_chunks // 2), base_chunks + 1):
            @pl.when(jt < my_chunks)
            def _(jt=jt):
                step(jt, jt % 2)

        plsc.subcore_barrier()
        pltpu.sync_copy(agg_sh.at[pl.ds(s * base_rows, base_rows)],
                        out_hbm.at[c, pl.ds(s * base_rows, base_rows)])

        @pl.when(s == NS - 1)
        def _():
            pltpu.sync_copy(agg_sh.at[pl.ds(NS * base_rows, tail_rows)],
                            out_hbm.at[c, pl.ds(NS * base_rows, tail_rows)])

    return k(node_features, mix, edge_src, edge_dst)


def _tc_final(node_self_out, aggs, W_out_scaled, cos_mix):
    Nn, DOUT = node_self_out.shape
    D = aggs[0].shape[2]
    BN = 1000
    assert Nn % BN == 0

    def body(nso_ref, *rest):
        agg_refs, w_ref, out_ref = rest[:-2], rest[-2], rest[-1]
        agg = agg_refs[0][0] + agg_refs[0][1]
        for a in agg_refs[1:]:
            agg = agg + a[0] + a[1]
        conv = jnp.dot(agg, w_ref[...], preferred_element_type=jnp.float32)
        out_ref[...] = cos_mix * nso_ref[...] + conv

    return pl.pallas_call(
        body,
        grid=(Nn // BN,),
        in_specs=[
            pl.BlockSpec((BN, DOUT), lambda i: (i, 0)),
        ] + [
            pl.BlockSpec((2, BN, D), lambda i: (0, i, 0)) for _ in aggs
        ] + [
            pl.BlockSpec((D, DOUT), lambda i: (0, 0)),
        ],
        out_specs=pl.BlockSpec((BN, DOUT), lambda i: (i, 0)),
        out_shape=jax.ShapeDtypeStruct((Nn, DOUT), jnp.float32),
    )(node_self_out, *aggs, W_out_scaled)


def kernel(node_input, edge_src, edge_dst, edge_attr, edge_scalar_attr,
           W_self, W_mlp1, W_mlp2, W_tp, W_out):
    H, D, DE = W_tp.shape
    NUM_NEIGHBORS = 32.0
    MIXING_ANGLE = 0.39269908169872414
    c = np.cos(MIXING_ANGLE)
    s = np.sin(MIXING_ANGLE)

    Wflat = (W_tp.transpose(0, 2, 1) / np.sqrt(H)).reshape(H * DE, D)
    W_out_scaled = W_out * (s / np.sqrt(NUM_NEIGHBORS))
    HW = 128
    W1P = jnp.pad(W_mlp1, ((0, 0), (0, HW - H)))
    W2R = jnp.pad(jnp.repeat(W_mlp2, DE, axis=1),
                  ((0, HW - H), (0, 0)))
    T = jnp.tile(jnp.eye(DE, dtype=jnp.float32), (1, H))

    node_features, node_self_out = _tc_self(node_input, W_self)
    E = edge_src.shape[0]
    NPH = 4
    EP = E // NPH
    partials = []
    for ph in range(NPH):
        lo = ph * EP
        mix_ph = _tc_mix(edge_attr.T, edge_scalar_attr.T, W1P, W2R, T, Wflat,
                         e_lo=lo, e_cnt=EP)
        partials.append(
            _sc_gather_mul_scatter(node_features, mix_ph,
                                   lax.slice(edge_src, (lo,), (lo + EP,)),
                                   lax.slice(edge_dst, (lo,), (lo + EP,))))
    return _tc_final(node_self_out, partials, W_out_scaled, c)
